# Initial kernel scaffold; baseline (speedup 1.0000x reference)
import jax
import jax.numpy as jnp
from jax.experimental import pallas as pl


def kernel(x, conv_w, conv_b, w1, b1, gamma, beta, w2, b2):
    raise NotImplementedError("write your pallas kernel here")


# speedup vs baseline: 1.1442x; 1.1442x over previous
import functools

import jax
import jax.numpy as jnp
from jax import lax
from jax.experimental import pallas as pl
from jax.experimental.pallas import tpu as pltpu

_J = 8


def _shift_lanes(v, off):
    n = v.shape[-1]
    k = off % n
    if k == 0:
        return v
    return jnp.concatenate([v[..., k:], v[..., :k]], axis=-1)


def _conv_lin1_kernel(x_ref, w8_ref, cb_ref, w1_ref, b1_ref, rep_ref, h_ref,
                      *, H, W, C, F, KH, KW):
    Gblk = x_ref.shape[0]
    HW = H * W
    Hd = w1_ref.shape[2]

    xb = x_ref[...].astype(jnp.bfloat16)

    lane = lax.broadcasted_iota(jnp.int32, (1, 1, HW), 2)
    yy = lane // W
    xx = lane - yy * W

    tiles = []
    for oy in range(-(KH // 2), KH - KH // 2):
        for ox in range(-(KW // 2), KW - KW // 2):
            m = ((yy + oy >= 0) & (yy + oy < H) &
                 (xx + ox >= 0) & (xx + ox < W))
            tiles.append(jnp.where(m, _shift_lanes(xb, oy * W + ox),
                                   jnp.bfloat16(0)))
    patch = jnp.concatenate(tiles, axis=1)

    w8 = w8_ref[...]
    cb = cb_ref[...]
    convs = []
    for g in range(Gblk):
        cg = jnp.dot(w8, patch[g],
                     preferred_element_type=jnp.float32)
        convs.append(jnp.maximum(cg + cb, 0.0))
    conv = jnp.stack(convs, axis=0)
    rep_ref[...] = conv

    bm = conv.reshape(Gblk * _J, F, HW).astype(jnp.bfloat16)
    w1 = w1_ref[...]
    h = jnp.zeros((Gblk * _J, Hd), jnp.float32)
    for f in range(F):
        h = h + jnp.dot(bm[:, f, :], w1[f],
                        preferred_element_type=jnp.float32)
    h_ref[...] = h + b1_ref[...]


def _bn_lin2_kernel(h_ref, g_ref, bt_ref, w2_ref, b2_ref, out_ref):
    h = h_ref[...]
    B = h.shape[0]
    s1 = jnp.sum(h, axis=0, keepdims=True)
    s2 = jnp.sum(h * h, axis=0, keepdims=True)
    mean = s1 * (1.0 / B)
    var = s2 * (1.0 / B) - mean * mean
    scale = g_ref[...] * lax.rsqrt(var + 1e-5)
    shift = bt_ref[...] - mean * scale
    hn = jnp.maximum(h * scale + shift, 0.0).astype(jnp.bfloat16)
    out_ref[...] = (jnp.dot(hn, w2_ref[...],
                            preferred_element_type=jnp.float32) + b2_ref[...])


def kernel(x, conv_w, conv_b, w1, b1, gamma, beta, w2, b2):
    B, C, H, W = x.shape
    F, _, KH, KW = conv_w.shape
    HW = H * W
    D, Hd = w1.shape
    P = w2.shape[1]
    J = _J
    G = B // J

    Gblk = min(16, G)
    while G % Gblk:
        Gblk -= 1

    x3 = x.reshape(G, J * C, HW)
    wtc = jnp.transpose(conv_w, (0, 2, 3, 1)).reshape(F, KH * KW, C)
    w8 = jnp.einsum('ij,ftc->iftjc', jnp.eye(J, dtype=conv_w.dtype), wtc)
    w8 = w8.reshape(J * F, KH * KW * J * C).astype(jnp.bfloat16)
    cb8 = jnp.tile(conv_b, (J,)).reshape(J * F, 1)
    w1p = w1.reshape(F, HW, Hd).astype(jnp.bfloat16)
    b1r = b1.reshape(1, Hd)

    body = functools.partial(_conv_lin1_kernel, H=H, W=W, C=C, F=F,
                             KH=KH, KW=KW)
    rep, h = pl.pallas_call(
        body,
        out_shape=(jax.ShapeDtypeStruct((G, J * F, HW), jnp.float32),
                   jax.ShapeDtypeStruct((B, Hd), jnp.float32)),
        grid=(G // Gblk,),
        in_specs=[
            pl.BlockSpec((Gblk, J * C, HW), lambda i: (i, 0, 0)),
            pl.BlockSpec((J * F, KH * KW * J * C), lambda i: (0, 0)),
            pl.BlockSpec((J * F, 1), lambda i: (0, 0)),
            pl.BlockSpec((F, HW, Hd), lambda i: (0, 0, 0)),
            pl.BlockSpec((1, Hd), lambda i: (0, 0)),
        ],
        out_specs=(
            pl.BlockSpec((Gblk, J * F, HW), lambda i: (i, 0, 0)),
            pl.BlockSpec((Gblk * J, Hd), lambda i: (i, 0)),
        ),
        compiler_params=pltpu.CompilerParams(
            dimension_semantics=("parallel",),
            vmem_limit_bytes=100 * 1024 * 1024,
        ),
    )(x3, w8, cb8, w1p, b1r)

    def full(shape):
        return pl.BlockSpec(shape, lambda: (0,) * len(shape))

    projection = pl.pallas_call(
        _bn_lin2_kernel,
        out_shape=jax.ShapeDtypeStruct((B, P), jnp.float32),
        in_specs=[full((B, Hd)), full((1, Hd)), full((1, Hd)),
                  full((Hd, P)), full((1, P))],
        out_specs=full((B, P)),
    )(h, gamma.reshape(1, Hd), beta.reshape(1, Hd),
      w2.astype(jnp.bfloat16), b2.reshape(1, P))

    return projection, rep.reshape(B, D)


# speedup vs baseline: 1.3478x; 1.1780x over previous
import functools

import jax
import jax.numpy as jnp
from jax import lax
from jax.experimental import pallas as pl
from jax.experimental.pallas import tpu as pltpu

_J = 8


def _shift_lanes(v, off):
    n = v.shape[-1]
    k = off % n
    if k == 0:
        return v
    return jnp.concatenate([v[..., k:], v[..., :k]], axis=-1)


def _conv_lin1_kernel(x_ref, w8_ref, cb_ref, w1_ref, b1_ref, rep_ref, h_ref,
                      *, H, W, C, F, KH, KW):
    Gblk = x_ref.shape[0]
    HW = H * W
    Hd = w1_ref.shape[2]

    xb = x_ref[...].astype(jnp.bfloat16)

    lane = lax.broadcasted_iota(jnp.int32, (1, 1, HW), 2)
    yy = lane // W
    xx = lane - yy * W

    tiles = []
    for oy in range(-(KH // 2), KH - KH // 2):
        for ox in range(-(KW // 2), KW - KW // 2):
            m = ((yy + oy >= 0) & (yy + oy < H) &
                 (xx + ox >= 0) & (xx + ox < W))
            tiles.append(jnp.where(m, _shift_lanes(xb, oy * W + ox),
                                   jnp.bfloat16(0)))
    patch = jnp.concatenate(tiles, axis=1)

    w8 = w8_ref[...]
    cb = cb_ref[...]
    JF = _J * F
    reps = []
    lin = []
    for g in range(Gblk):
        cg = jnp.dot(w8, patch[g],
                     preferred_element_type=jnp.float32)
        act = jnp.maximum(cg + cb, 0.0)
        reps.append(act[:JF])
        lin.append(act[JF:].astype(jnp.bfloat16))
    rep_ref[...] = jnp.stack(reps, axis=0)

    lb = jnp.stack(lin, axis=0)
    w1 = w1_ref[...]
    h = jnp.zeros((Gblk * _J, Hd), jnp.float32)
    for f in range(F):
        lhs = lb[:, f * _J:(f + 1) * _J, :].reshape(Gblk * _J, HW)
        h = h + jnp.dot(lhs, w1[f],
                        preferred_element_type=jnp.float32)
    h_ref[...] = h + b1_ref[...]


def _bn_lin2_kernel(h_ref, g_ref, bt_ref, w2_ref, b2_ref, out_ref):
    h = h_ref[...]
    B = h.shape[0]
    s1 = jnp.sum(h, axis=0, keepdims=True)
    s2 = jnp.sum(h * h, axis=0, keepdims=True)
    mean = s1 * (1.0 / B)
    var = s2 * (1.0 / B) - mean * mean
    scale = g_ref[...] * lax.rsqrt(var + 1e-5)
    shift = bt_ref[...] - mean * scale
    hn = jnp.maximum(h * scale + shift, 0.0).astype(jnp.bfloat16)
    out_ref[...] = (jnp.dot(hn, w2_ref[...],
                            preferred_element_type=jnp.float32) + b2_ref[...])


def kernel(x, conv_w, conv_b, w1, b1, gamma, beta, w2, b2):
    B, C, H, W = x.shape
    F, _, KH, KW = conv_w.shape
    HW = H * W
    D, Hd = w1.shape
    P = w2.shape[1]
    J = _J
    G = B // J

    Gblk = min(16, G)
    while G % Gblk:
        Gblk -= 1

    x3 = x.reshape(G, J * C, HW)
    wtc = jnp.transpose(conv_w, (0, 2, 3, 1)).reshape(F, KH * KW, C)
    eyeJ = jnp.eye(J, dtype=conv_w.dtype)
    w8a = jnp.einsum('ij,ftc->iftjc', eyeJ, wtc).reshape(J * F, KH * KW * J * C)
    w8b = jnp.einsum('ij,ftc->fitjc', eyeJ, wtc).reshape(J * F, KH * KW * J * C)
    w8 = jnp.concatenate([w8a, w8b], axis=0).astype(jnp.bfloat16)
    cb8 = jnp.concatenate([jnp.tile(conv_b, (J,)),
                           jnp.repeat(conv_b, J)]).reshape(2 * J * F, 1)
    w1p = w1.reshape(F, HW, Hd).astype(jnp.bfloat16)
    b1r = b1.reshape(1, Hd)

    body = functools.partial(_conv_lin1_kernel, H=H, W=W, C=C, F=F,
                             KH=KH, KW=KW)
    rep, h = pl.pallas_call(
        body,
        out_shape=(jax.ShapeDtypeStruct((G, J * F, HW), jnp.float32),
                   jax.ShapeDtypeStruct((B, Hd), jnp.float32)),
        grid=(G // Gblk,),
        in_specs=[
            pl.BlockSpec((Gblk, J * C, HW), lambda i: (i, 0, 0)),
            pl.BlockSpec((2 * J * F, KH * KW * J * C), lambda i: (0, 0)),
            pl.BlockSpec((2 * J * F, 1), lambda i: (0, 0)),
            pl.BlockSpec((F, HW, Hd), lambda i: (0, 0, 0)),
            pl.BlockSpec((1, Hd), lambda i: (0, 0)),
        ],
        out_specs=(
            pl.BlockSpec((Gblk, J * F, HW), lambda i: (i, 0, 0)),
            pl.BlockSpec((Gblk * J, Hd), lambda i: (i, 0)),
        ),
        compiler_params=pltpu.CompilerParams(
            dimension_semantics=("parallel",),
            vmem_limit_bytes=100 * 1024 * 1024,
        ),
    )(x3, w8, cb8, w1p, b1r)

    def full(shape):
        return pl.BlockSpec(shape, lambda: (0,) * len(shape))

    projection = pl.pallas_call(
        _bn_lin2_kernel,
        out_shape=jax.ShapeDtypeStruct((B, P), jnp.float32),
        in_specs=[full((B, Hd)), full((1, Hd)), full((1, Hd)),
                  full((Hd, P)), full((1, P))],
        out_specs=full((B, P)),
    )(h, gamma.reshape(1, Hd), beta.reshape(1, Hd),
      w2.astype(jnp.bfloat16), b2.reshape(1, P))

    return projection, rep.reshape(B, D)


# speedup vs baseline: 2.0014x; 1.4849x over previous
import functools

import jax
import jax.numpy as jnp
from jax import lax
from jax.experimental import pallas as pl
from jax.experimental.pallas import tpu as pltpu

_J = 8


def _shift_lanes(v, off):
    n = v.shape[-1]
    k = off % n
    if k == 0:
        return v
    return jnp.concatenate([v[..., k:], v[..., :k]], axis=-1)


def _conv_lin1_kernel(x_ref, w8_ref, cb_ref, w1_ref, b1_ref, rep_ref, h_ref,
                      *, H, W, C, F, KH, KW):
    HW = H * W
    Bblk = x_ref.shape[0]
    Gblk = Bblk // _J
    Hd = w1_ref.shape[1]

    xb = x_ref[...].astype(jnp.bfloat16).reshape(Gblk, _J * C, HW)

    lane = lax.broadcasted_iota(jnp.int32, (1, 1, HW), 2)
    yy = lane // W
    xx = lane - yy * W

    tiles = []
    for oy in range(-(KH // 2), KH - KH // 2):
        for ox in range(-(KW // 2), KW - KW // 2):
            m = ((yy + oy >= 0) & (yy + oy < H) &
                 (xx + ox >= 0) & (xx + ox < W))
            tiles.append(jnp.where(m, _shift_lanes(xb, oy * W + ox),
                                   jnp.bfloat16(0)))
    patch = jnp.concatenate(tiles, axis=1)

    w8 = w8_ref[...]
    cb = cb_ref[...]
    acts = []
    for g in range(Gblk):
        cg = jnp.dot(w8, patch[g],
                     preferred_element_type=jnp.float32)
        acts.append(jnp.maximum(cg + cb, 0.0))
    conv = jnp.stack(acts, axis=0)

    repb = conv.reshape(Bblk, F * HW)
    rep_ref[...] = repb

    h = jnp.zeros((Bblk, Hd), jnp.float32)
    for f in range(F):
        lhs = repb[:, f * HW:(f + 1) * HW].astype(jnp.bfloat16)
        h = h + jnp.dot(lhs, w1_ref[f * HW:(f + 1) * HW, :],
                        preferred_element_type=jnp.float32)
    h_ref[...] = h + b1_ref[...]


def _bn_lin2_kernel(h_ref, g_ref, bt_ref, w2_ref, b2_ref, out_ref):
    h = h_ref[...]
    B = h.shape[0]
    s1 = jnp.sum(h, axis=0, keepdims=True)
    s2 = jnp.sum(h * h, axis=0, keepdims=True)
    mean = s1 * (1.0 / B)
    var = s2 * (1.0 / B) - mean * mean
    scale = g_ref[...] * lax.rsqrt(var + 1e-5)
    shift = bt_ref[...] - mean * scale
    hn = jnp.maximum(h * scale + shift, 0.0).astype(jnp.bfloat16)
    out_ref[...] = (jnp.dot(hn, w2_ref[...],
                            preferred_element_type=jnp.float32) + b2_ref[...])


def kernel(x, conv_w, conv_b, w1, b1, gamma, beta, w2, b2):
    B, C, H, W = x.shape
    F, _, KH, KW = conv_w.shape
    HW = H * W
    D, Hd = w1.shape
    P = w2.shape[1]
    J = _J

    Bblk = min(128, B)
    while B % Bblk or Bblk % J:
        Bblk -= 1

    wtc = jnp.transpose(conv_w, (0, 2, 3, 1)).reshape(F, KH * KW, C)
    eyeJ = jnp.eye(J, dtype=conv_w.dtype)
    w8 = jnp.einsum('ij,ftc->iftjc', eyeJ, wtc)
    w8 = w8.reshape(J * F, KH * KW * J * C).astype(jnp.bfloat16)
    cb8 = jnp.tile(conv_b, (J,)).reshape(J * F, 1)
    w1b = w1.astype(jnp.bfloat16)
    b1r = b1.reshape(1, Hd)

    body = functools.partial(_conv_lin1_kernel, H=H, W=W, C=C, F=F,
                             KH=KH, KW=KW)
    rep, h = pl.pallas_call(
        body,
        out_shape=(jax.ShapeDtypeStruct((B, D), jnp.float32),
                   jax.ShapeDtypeStruct((B, Hd), jnp.float32)),
        grid=(B // Bblk,),
        in_specs=[
            pl.BlockSpec((Bblk, C, H, W), lambda i: (i, 0, 0, 0)),
            pl.BlockSpec((J * F, KH * KW * J * C), lambda i: (0, 0)),
            pl.BlockSpec((J * F, 1), lambda i: (0, 0)),
            pl.BlockSpec((D, Hd), lambda i: (0, 0)),
            pl.BlockSpec((1, Hd), lambda i: (0, 0)),
        ],
        out_specs=(
            pl.BlockSpec((Bblk, D), lambda i: (i, 0)),
            pl.BlockSpec((Bblk, Hd), lambda i: (i, 0)),
        ),
        compiler_params=pltpu.CompilerParams(
            dimension_semantics=("parallel",),
            vmem_limit_bytes=100 * 1024 * 1024,
        ),
    )(x, w8, cb8, w1b, b1r)

    def full(shape):
        return pl.BlockSpec(shape, lambda: (0,) * len(shape))

    projection = pl.pallas_call(
        _bn_lin2_kernel,
        out_shape=jax.ShapeDtypeStruct((B, P), jnp.float32),
        in_specs=[full((B, Hd)), full((1, Hd)), full((1, Hd)),
                  full((Hd, P)), full((1, P))],
        out_specs=full((B, P)),
    )(h, gamma.reshape(1, Hd), beta.reshape(1, Hd),
      w2.astype(jnp.bfloat16), b2.reshape(1, P))

    return projection, rep


# speedup vs baseline: 2.7128x; 1.3554x over previous
import functools

import jax
import jax.numpy as jnp
from jax import lax
from jax.experimental import pallas as pl
from jax.experimental.pallas import tpu as pltpu

_J = 8


def _shift_lanes(v, off):
    n = v.shape[-1]
    k = off % n
    if k == 0:
        return v
    return jnp.concatenate([v[..., k:], v[..., :k]], axis=-1)


def _conv_lin1_kernel(x_ref, w8_ref, cb_ref, w1_ref, b1_ref, rep_ref, h_ref,
                      *, H, W, C, F, KH, KW):
    HW = H * W
    Bblk = x_ref.shape[3]
    Gblk = Bblk // _J
    Hd = w1_ref.shape[1]

    xq = jnp.transpose(x_ref[...].astype(jnp.bfloat16), (3, 0, 1, 2))
    xb = xq.reshape(Gblk, _J * C, HW)

    lane = lax.broadcasted_iota(jnp.int32, (1, 1, HW), 2)
    yy = lane // W
    xx = lane - yy * W

    tiles = []
    for oy in range(-(KH // 2), KH - KH // 2):
        for ox in range(-(KW // 2), KW - KW // 2):
            m = ((yy + oy >= 0) & (yy + oy < H) &
                 (xx + ox >= 0) & (xx + ox < W))
            tiles.append(jnp.where(m, _shift_lanes(xb, oy * W + ox),
                                   jnp.bfloat16(0)))
    patch = jnp.concatenate(tiles, axis=1)

    w8 = w8_ref[...]
    cb = cb_ref[...]
    acts = []
    for g in range(Gblk):
        cg = jnp.dot(w8, patch[g],
                     preferred_element_type=jnp.float32)
        acts.append(jnp.maximum(cg + cb, 0.0))
    conv = jnp.stack(acts, axis=0)

    repb = conv.reshape(Bblk, F * HW)
    rep_ref[...] = repb

    h = jnp.zeros((Bblk, Hd), jnp.float32)
    for f in range(F):
        lhs = repb[:, f * HW:(f + 1) * HW].astype(jnp.bfloat16)
        h = h + jnp.dot(lhs, w1_ref[f * HW:(f + 1) * HW, :],
                        preferred_element_type=jnp.float32)
    h_ref[...] = h + b1_ref[...]


def _bn_lin2_kernel(h_ref, g_ref, bt_ref, w2_ref, b2_ref, out_ref):
    h = h_ref[...]
    B = h.shape[0]
    s1 = jnp.sum(h, axis=0, keepdims=True)
    s2 = jnp.sum(h * h, axis=0, keepdims=True)
    mean = s1 * (1.0 / B)
    var = s2 * (1.0 / B) - mean * mean
    scale = g_ref[...] * lax.rsqrt(var + 1e-5)
    shift = bt_ref[...] - mean * scale
    hn = jnp.maximum(h * scale + shift, 0.0).astype(jnp.bfloat16)
    out_ref[...] = (jnp.dot(hn, w2_ref[...],
                            preferred_element_type=jnp.float32) + b2_ref[...])


def kernel(x, conv_w, conv_b, w1, b1, gamma, beta, w2, b2):
    B, C, H, W = x.shape
    F, _, KH, KW = conv_w.shape
    HW = H * W
    D, Hd = w1.shape
    P = w2.shape[1]
    J = _J

    Bblk = min(128, B)
    while B % Bblk or Bblk % J:
        Bblk -= 1

    wtc = jnp.transpose(conv_w, (0, 2, 3, 1)).reshape(F, KH * KW, C)
    eyeJ = jnp.eye(J, dtype=conv_w.dtype)
    w8 = jnp.einsum('ij,ftc->iftjc', eyeJ, wtc)
    w8 = w8.reshape(J * F, KH * KW * J * C).astype(jnp.bfloat16)
    cb8 = jnp.tile(conv_b, (J,)).reshape(J * F, 1)
    w1b = w1.astype(jnp.bfloat16)
    b1r = b1.reshape(1, Hd)

    body = functools.partial(_conv_lin1_kernel, H=H, W=W, C=C, F=F,
                             KH=KH, KW=KW)
    rep, h = pl.pallas_call(
        body,
        out_shape=(jax.ShapeDtypeStruct((B, D), jnp.float32),
                   jax.ShapeDtypeStruct((B, Hd), jnp.float32)),
        grid=(B // Bblk,),
        in_specs=[
            pl.BlockSpec((C, H, W, Bblk), lambda i: (0, 0, 0, i)),
            pl.BlockSpec((J * F, KH * KW * J * C), lambda i: (0, 0)),
            pl.BlockSpec((J * F, 1), lambda i: (0, 0)),
            pl.BlockSpec((D, Hd), lambda i: (0, 0)),
            pl.BlockSpec((1, Hd), lambda i: (0, 0)),
        ],
        out_specs=(
            pl.BlockSpec((Bblk, D), lambda i: (i, 0)),
            pl.BlockSpec((Bblk, Hd), lambda i: (i, 0)),
        ),
        compiler_params=pltpu.CompilerParams(
            dimension_semantics=("parallel",),
            vmem_limit_bytes=100 * 1024 * 1024,
        ),
    )(jnp.transpose(x, (1, 2, 3, 0)), w8, cb8, w1b, b1r)

    def full(shape):
        return pl.BlockSpec(shape, lambda: (0,) * len(shape))

    projection = pl.pallas_call(
        _bn_lin2_kernel,
        out_shape=jax.ShapeDtypeStruct((B, P), jnp.float32),
        in_specs=[full((B, Hd)), full((1, Hd)), full((1, Hd)),
                  full((Hd, P)), full((1, P))],
        out_specs=full((B, P)),
    )(h, gamma.reshape(1, Hd), beta.reshape(1, Hd),
      w2.astype(jnp.bfloat16), b2.reshape(1, P))

    return projection, rep


# speedup vs baseline: 3.8581x; 1.4222x over previous
import functools

import jax
import jax.numpy as jnp
from jax import lax
from jax.experimental import pallas as pl
from jax.experimental.pallas import tpu as pltpu

_J = 8


def _shift_lanes(v, off):
    n = v.shape[-1]
    k = off % n
    if k == 0:
        return v
    return jnp.concatenate([v[..., k:], v[..., :k]], axis=-1)


def _conv_lin1_kernel(x_ref, w8_ref, w1_ref, b1_ref, rep_ref, h_ref,
                      *, H, W, C, F, KH, KW):
    HW = H * W
    Bblk = x_ref.shape[3]
    Gblk = Bblk // _J
    Hd = w1_ref.shape[1]

    xm = x_ref[...].reshape(C, HW, Bblk)
    xt = jnp.transpose(xm, (0, 2, 1))
    xq = jnp.transpose(xt.reshape(C, Gblk, _J, HW), (1, 0, 2, 3))
    xb = xq.reshape(Gblk, C * _J, HW).astype(jnp.bfloat16)

    lane = lax.broadcasted_iota(jnp.int32, (1, 1, HW), 2)
    yy = lane // W
    xx = lane - yy * W

    tiles = []
    for oy in range(-(KH // 2), KH - KH // 2):
        for ox in range(-(KW // 2), KW - KW // 2):
            m = ((yy + oy >= 0) & (yy + oy < H) &
                 (xx + ox >= 0) & (xx + ox < W))
            tiles.append(jnp.where(m, _shift_lanes(xb, oy * W + ox),
                                   jnp.bfloat16(0)))
    tiles.append(jnp.ones((Gblk, _J, HW), jnp.bfloat16))
    patch = jnp.concatenate(tiles, axis=1)

    w8 = w8_ref[...]
    acts = []
    for g in range(Gblk):
        cg = jnp.dot(w8, patch[g],
                     preferred_element_type=jnp.float32)
        acts.append(jnp.maximum(cg, 0.0))
    conv = jnp.stack(acts, axis=0)

    repb = conv.reshape(Bblk, F * HW)
    rep_ref[...] = repb

    h = jnp.zeros((Bblk, Hd), jnp.float32)
    for f in range(F):
        lhs = repb[:, f * HW:(f + 1) * HW].astype(jnp.bfloat16)
        h = h + jnp.dot(lhs, w1_ref[f * HW:(f + 1) * HW, :],
                        preferred_element_type=jnp.float32)
    h_ref[...] = h + b1_ref[...]


def _bn_lin2_kernel(h_ref, g_ref, bt_ref, w2_ref, b2_ref, out_ref):
    h = h_ref[...]
    B = h.shape[0]
    s1 = jnp.sum(h, axis=0, keepdims=True)
    s2 = jnp.sum(h * h, axis=0, keepdims=True)
    mean = s1 * (1.0 / B)
    var = s2 * (1.0 / B) - mean * mean
    scale = g_ref[...] * lax.rsqrt(var + 1e-5)
    shift = bt_ref[...] - mean * scale
    hn = jnp.maximum(h * scale + shift, 0.0).astype(jnp.bfloat16)
    out_ref[...] = (jnp.dot(hn, w2_ref[...],
                            preferred_element_type=jnp.float32) + b2_ref[...])


def kernel(x, conv_w, conv_b, w1, b1, gamma, beta, w2, b2):
    B, C, H, W = x.shape
    F, _, KH, KW = conv_w.shape
    HW = H * W
    D, Hd = w1.shape
    P = w2.shape[1]
    J = _J

    Bblk = min(128, B)
    while B % Bblk or Bblk % J:
        Bblk -= 1

    wtc = jnp.transpose(conv_w, (0, 2, 3, 1)).reshape(F, KH * KW, C)
    eyeJ = jnp.eye(J, dtype=conv_w.dtype)
    w8 = jnp.einsum('ij,ftc->iftcj', eyeJ, wtc)
    w8 = w8.reshape(J * F, KH * KW * J * C)
    bias_blk = jnp.pad(jnp.tile(conv_b, (J,)).reshape(J * F, 1),
                       ((0, 0), (0, J - 1)))
    w8 = jnp.concatenate([w8, bias_blk], axis=1).astype(jnp.bfloat16)
    w1b = w1.astype(jnp.bfloat16)
    b1r = b1.reshape(1, Hd)

    body = functools.partial(_conv_lin1_kernel, H=H, W=W, C=C, F=F,
                             KH=KH, KW=KW)
    rep, h = pl.pallas_call(
        body,
        out_shape=(jax.ShapeDtypeStruct((B, D), jnp.float32),
                   jax.ShapeDtypeStruct((B, Hd), jnp.float32)),
        grid=(B // Bblk,),
        in_specs=[
            pl.BlockSpec((C, H, W, Bblk), lambda i: (0, 0, 0, i)),
            pl.BlockSpec((J * F, KH * KW * J * C + J), lambda i: (0, 0)),
            pl.BlockSpec((D, Hd), lambda i: (0, 0)),
            pl.BlockSpec((1, Hd), lambda i: (0, 0)),
        ],
        out_specs=(
            pl.BlockSpec((Bblk, D), lambda i: (i, 0)),
            pl.BlockSpec((Bblk, Hd), lambda i: (i, 0)),
        ),
        compiler_params=pltpu.CompilerParams(
            dimension_semantics=("parallel",),
            vmem_limit_bytes=100 * 1024 * 1024,
        ),
    )(jnp.transpose(x, (1, 2, 3, 0)), w8, w1b, b1r)

    def full(shape):
        return pl.BlockSpec(shape, lambda: (0,) * len(shape))

    projection = pl.pallas_call(
        _bn_lin2_kernel,
        out_shape=jax.ShapeDtypeStruct((B, P), jnp.float32),
        in_specs=[full((B, Hd)), full((1, Hd)), full((1, Hd)),
                  full((Hd, P)), full((1, P))],
        out_specs=full((B, P)),
    )(h, gamma.reshape(1, Hd), beta.reshape(1, Hd),
      w2.astype(jnp.bfloat16), b2.reshape(1, P))

    return projection, rep


# speedup vs baseline: 4.1618x; 1.0787x over previous
import functools

import jax
import jax.numpy as jnp
from jax import lax
from jax.experimental import pallas as pl
from jax.experimental.pallas import tpu as pltpu

_J = 8


def _shift_lanes(v, off):
    n = v.shape[-1]
    k = off % n
    if k == 0:
        return v
    return jnp.concatenate([v[..., k:], v[..., :k]], axis=-1)


def _conv_lin1_kernel(x_ref, w8_ref, w1_ref, b1_ref, rep_ref, h_ref,
                      *, H, W, C, F, KH, KW):
    HW = H * W
    Bblk = x_ref.shape[3]
    Gblk = Bblk // _J
    Hd = w1_ref.shape[1]

    xm = x_ref[...].reshape(C, HW, Bblk)
    xt = jnp.transpose(xm, (0, 2, 1))
    xq = jnp.transpose(xt.reshape(C, Gblk, _J, HW), (1, 0, 2, 3))
    xb = xq.reshape(Gblk, C * _J, HW).astype(jnp.bfloat16)

    lane = lax.broadcasted_iota(jnp.int32, (1, 1, HW), 2)
    yy = lane // W
    xx = lane - yy * W

    tiles = []
    for oy in range(-(KH // 2), KH - KH // 2):
        for ox in range(-(KW // 2), KW - KW // 2):
            m = ((yy + oy >= 0) & (yy + oy < H) &
                 (xx + ox >= 0) & (xx + ox < W))
            tiles.append(jnp.where(m, _shift_lanes(xb, oy * W + ox),
                                   jnp.bfloat16(0)))
    tiles.append(jnp.ones((Gblk, _J, HW), jnp.bfloat16))
    patch = jnp.concatenate(tiles, axis=1)

    w8 = w8_ref[...]
    acts = []
    for g in range(Gblk):
        cg = jnp.dot(w8, patch[g],
                     preferred_element_type=jnp.float32)
        acts.append(jnp.maximum(cg, 0.0))
    conv = jnp.stack(acts, axis=0)

    repb = conv.reshape(Bblk, F * HW)
    rep_ref[...] = repb

    h = jnp.zeros((Bblk, Hd), jnp.float32)
    for f in range(F):
        h = h + jnp.dot(repb[:, f * HW:(f + 1) * HW],
                        w1_ref[f * HW:(f + 1) * HW, :],
                        preferred_element_type=jnp.float32)
    h_ref[...] = h + b1_ref[...]


def _bn_lin2_kernel(h_ref, g_ref, bt_ref, w2_ref, b2_ref, out_ref):
    h = h_ref[...]
    B = h.shape[0]
    s1 = jnp.sum(h, axis=0, keepdims=True)
    s2 = jnp.sum(h * h, axis=0, keepdims=True)
    mean = s1 * (1.0 / B)
    var = s2 * (1.0 / B) - mean * mean
    scale = g_ref[...] * lax.rsqrt(var + 1e-5)
    shift = bt_ref[...] - mean * scale
    hn = jnp.maximum(h * scale + shift, 0.0)
    out_ref[...] = (jnp.dot(hn, w2_ref[...],
                            preferred_element_type=jnp.float32) + b2_ref[...])


def kernel(x, conv_w, conv_b, w1, b1, gamma, beta, w2, b2):
    B, C, H, W = x.shape
    F, _, KH, KW = conv_w.shape
    HW = H * W
    D, Hd = w1.shape
    P = w2.shape[1]
    J = _J

    Bblk = min(128, B)
    while B % Bblk or Bblk % J:
        Bblk -= 1

    wtc = jnp.transpose(conv_w, (0, 2, 3, 1)).reshape(F, KH * KW, C)
    eyeJ = jnp.eye(J, dtype=conv_w.dtype)
    w8 = jnp.einsum('ij,ftc->iftcj', eyeJ, wtc)
    w8 = w8.reshape(J * F, KH * KW * J * C)
    bias_blk = jnp.pad(jnp.tile(conv_b, (J,)).reshape(J * F, 1),
                       ((0, 0), (0, J - 1)))
    w8 = jnp.concatenate([w8, bias_blk], axis=1).astype(jnp.bfloat16)
    b1r = b1.reshape(1, Hd)

    body = functools.partial(_conv_lin1_kernel, H=H, W=W, C=C, F=F,
                             KH=KH, KW=KW)
    rep, h = pl.pallas_call(
        body,
        out_shape=(jax.ShapeDtypeStruct((B, D), jnp.float32),
                   jax.ShapeDtypeStruct((B, Hd), jnp.float32)),
        grid=(B // Bblk,),
        in_specs=[
            pl.BlockSpec((C, H, W, Bblk), lambda i: (0, 0, 0, i)),
            pl.BlockSpec((J * F, KH * KW * J * C + J), lambda i: (0, 0)),
            pl.BlockSpec((D, Hd), lambda i: (0, 0)),
            pl.BlockSpec((1, Hd), lambda i: (0, 0)),
        ],
        out_specs=(
            pl.BlockSpec((Bblk, D), lambda i: (i, 0)),
            pl.BlockSpec((Bblk, Hd), lambda i: (i, 0)),
        ),
        compiler_params=pltpu.CompilerParams(
            dimension_semantics=("parallel",),
            vmem_limit_bytes=100 * 1024 * 1024,
        ),
    )(jnp.transpose(x, (1, 2, 3, 0)), w8, w1, b1r)

    def full(shape):
        return pl.BlockSpec(shape, lambda: (0,) * len(shape))

    projection = pl.pallas_call(
        _bn_lin2_kernel,
        out_shape=jax.ShapeDtypeStruct((B, P), jnp.float32),
        in_specs=[full((B, Hd)), full((1, Hd)), full((1, Hd)),
                  full((Hd, P)), full((1, P))],
        out_specs=full((B, P)),
    )(h, gamma.reshape(1, Hd), beta.reshape(1, Hd),
      w2, b2.reshape(1, P))

    return projection, rep


# speedup vs baseline: 4.1718x; 1.0024x over previous
import functools

import jax
import jax.numpy as jnp
from jax import lax
from jax.experimental import pallas as pl
from jax.experimental.pallas import tpu as pltpu

_J = 8


def _shift_lanes(v, off):
    n = v.shape[-1]
    k = off % n
    if k == 0:
        return v
    return jnp.concatenate([v[..., k:], v[..., :k]], axis=-1)


def _conv_lin1_kernel(x_ref, w8_ref, w1_ref, b1_ref, rep_ref, h_ref,
                      *, H, W, C, F, KH, KW):
    HW = H * W
    Bblk = x_ref.shape[3]
    Gblk = Bblk // _J
    Hd = w1_ref.shape[1]

    xm = x_ref[...].reshape(C, HW, Bblk)
    xt = jnp.transpose(xm, (0, 2, 1))
    xq = jnp.transpose(xt.reshape(C, Gblk, _J, HW), (1, 0, 2, 3))
    xb = xq.reshape(Gblk, C * _J, HW).astype(jnp.bfloat16)

    lane = lax.broadcasted_iota(jnp.int32, (1, 1, HW), 2)
    yy = lane // W
    xx = lane - yy * W

    tiles = []
    for oy in range(-(KH // 2), KH - KH // 2):
        for ox in range(-(KW // 2), KW - KW // 2):
            m = ((yy + oy >= 0) & (yy + oy < H) &
                 (xx + ox >= 0) & (xx + ox < W))
            tiles.append(jnp.where(m, _shift_lanes(xb, oy * W + ox),
                                   jnp.bfloat16(0)))
    tiles.append(jnp.ones((Gblk, _J, HW), jnp.bfloat16))
    patch = jnp.concatenate(tiles, axis=1)

    w8 = w8_ref[...]
    acts = []
    for g in range(Gblk):
        cg = jnp.dot(w8, patch[g],
                     preferred_element_type=jnp.float32)
        acts.append(jnp.maximum(cg, 0.0))
    conv = jnp.stack(acts, axis=0)

    repb = conv.reshape(Bblk, F * HW)
    rep_ref[...] = repb

    h = jnp.zeros((Bblk, Hd), jnp.float32)
    for f in range(F):
        h = h + jnp.dot(repb[:, f * HW:(f + 1) * HW],
                        w1_ref[f * HW:(f + 1) * HW, :],
                        preferred_element_type=jnp.float32)
    h_ref[...] = h + b1_ref[...]


def _bn_lin2_kernel(h_ref, g_ref, bt_ref, w2_ref, b2_ref, out_ref):
    h = h_ref[...]
    B = h.shape[0]
    s1 = jnp.sum(h, axis=0, keepdims=True)
    s2 = jnp.sum(h * h, axis=0, keepdims=True)
    mean = s1 * (1.0 / B)
    var = s2 * (1.0 / B) - mean * mean
    scale = g_ref[...] * lax.rsqrt(var + 1e-5)
    shift = bt_ref[...] - mean * scale
    hn = jnp.maximum(h * scale + shift, 0.0)
    out_ref[...] = (jnp.dot(hn, w2_ref[...],
                            preferred_element_type=jnp.float32) + b2_ref[...])


def kernel(x, conv_w, conv_b, w1, b1, gamma, beta, w2, b2):
    B, C, H, W = x.shape
    F, _, KH, KW = conv_w.shape
    HW = H * W
    D, Hd = w1.shape
    P = w2.shape[1]
    J = _J

    Bblk = min(256, B)
    while B % Bblk or Bblk % J:
        Bblk -= 1

    wtc = jnp.transpose(conv_w, (0, 2, 3, 1)).reshape(F, KH * KW, C)
    eyeJ = jnp.eye(J, dtype=conv_w.dtype)
    w8 = jnp.einsum('ij,ftc->iftcj', eyeJ, wtc)
    w8 = w8.reshape(J * F, KH * KW * J * C)
    bias_blk = jnp.pad(jnp.tile(conv_b, (J,)).reshape(J * F, 1),
                       ((0, 0), (0, J - 1)))
    w8 = jnp.concatenate([w8, bias_blk], axis=1).astype(jnp.bfloat16)
    b1r = b1.reshape(1, Hd)

    body = functools.partial(_conv_lin1_kernel, H=H, W=W, C=C, F=F,
                             KH=KH, KW=KW)
    rep, h = pl.pallas_call(
        body,
        out_shape=(jax.ShapeDtypeStruct((B, D), jnp.float32),
                   jax.ShapeDtypeStruct((B, Hd), jnp.float32)),
        grid=(B // Bblk,),
        in_specs=[
            pl.BlockSpec((C, H, W, Bblk), lambda i: (0, 0, 0, i)),
            pl.BlockSpec((J * F, KH * KW * J * C + J), lambda i: (0, 0)),
            pl.BlockSpec((D, Hd), lambda i: (0, 0)),
            pl.BlockSpec((1, Hd), lambda i: (0, 0)),
        ],
        out_specs=(
            pl.BlockSpec((Bblk, D), lambda i: (i, 0)),
            pl.BlockSpec((Bblk, Hd), lambda i: (i, 0)),
        ),
        compiler_params=pltpu.CompilerParams(
            dimension_semantics=("parallel",),
            vmem_limit_bytes=100 * 1024 * 1024,
        ),
    )(jnp.transpose(x, (1, 2, 3, 0)), w8, w1, b1r)

    def full(shape):
        return pl.BlockSpec(shape, lambda: (0,) * len(shape))

    projection = pl.pallas_call(
        _bn_lin2_kernel,
        out_shape=jax.ShapeDtypeStruct((B, P), jnp.float32),
        in_specs=[full((B, Hd)), full((1, Hd)), full((1, Hd)),
                  full((Hd, P)), full((1, P))],
        out_specs=full((B, P)),
    )(h, gamma.reshape(1, Hd), beta.reshape(1, Hd),
      w2, b2.reshape(1, P))

    return projection, rep
